# Initial kernel scaffold; baseline (speedup 1.0000x reference)
import jax
import jax.numpy as jnp
from jax.experimental import pallas as pl


def kernel(x, edge_index, edge_weight, batch, W1, a_src1, a_dst1, b1, W2, a_src2, a_dst2, b2, Wlin, blin, Wcls, bcls):
    raise NotImplementedError("write your pallas kernel here")


# speedup vs baseline: 12.9422x; 12.9422x over previous
import dataclasses
import functools

import jax
import jax.numpy as jnp
from jax import lax
from jax.experimental import pallas as pl
from jax.experimental.pallas import tpu as pltpu
from jax.experimental.pallas import tpu_sc as plsc

N = 10000
E = 320000
D = 128
H = 64
G = 64
C = 10

NC = 2
NS = 16
NW = NC * NS
EC = E // NW
EB = 80
NB = EC // EB
CB = 1
WD = 80
RPT = N // NS

_ROW_BLK = 1000
_NBLK = N // _ROW_BLK


def _compiler_params_sc():
    cp = pltpu.CompilerParams(use_tc_tiling_on_sc=False)
    if "needs_layout_passes" in pltpu.CompilerParams.__dataclass_fields__:
        cp = dataclasses.replace(cp, needs_layout_passes=False)
    return cp


def _sc_edge_body(htab_hbm, sd_hbm, src_hbm, dst_hbm, ew_hbm, out_hbm,
                  acc, sd_v, src_v, dst_v, ew_v, rows, sem, gsem):
    c = lax.axis_index("c")
    s = lax.axis_index("s")
    wid = s * NC + c

    pltpu.sync_copy(sd_hbm, sd_v)

    @pl.loop(0, EB)
    def _(r):
        @pl.loop(0, WD, step=16)
        def _(cc):
            rows[r, pl.ds(cc, 16)] = jnp.zeros((16,), jnp.float32)

    @pl.loop(0, RPT // 25)
    def _(k):
        pltpu.sync_copy(rows.at[pl.ds(0, 25), :],
                        acc.at[pl.ds(s * RPT + k * 25, 25), :])

    plsc.subcore_barrier()


    @pl.loop(0, NB)
    def _(bb):
        pltpu.sync_copy(src_hbm.at[wid, pl.ds(bb, CB), :], src_v)
        pltpu.sync_copy(dst_hbm.at[wid, pl.ds(bb, CB), :], dst_v)
        pltpu.sync_copy(ew_hbm.at[wid, pl.ds(bb, CB), :], ew_v)

        @pl.loop(0, CB)
        def _(b):
            pltpu.async_copy(htab_hbm.at[src_v.at[b]], rows, gsem).wait()

            @pl.loop(0, EB, step=16)
            def _(e0):
                src16 = src_v[b, pl.ds(e0, 16)]
                dst16 = dst_v[b, pl.ds(e0, 16)]
                s16 = plsc.load_gather(sd_v, [src16 * 2])
                d16 = plsc.load_gather(sd_v, [dst16 * 2 + 1])
                a = s16 + d16
                a = jnp.where(a >= 0, a, a * jnp.float32(0.2))
                ex = jnp.exp(a)
                g = ex * ew_v[b, pl.ds(e0, 16)]
                for k in range(16):
                    gk = g[k]
                    exk = ex[k]
                    j = e0 + k
                    rows[j, pl.ds(0, 16)] = rows[j, pl.ds(0, 16)] * gk
                    rows[j, pl.ds(16, 16)] = rows[j, pl.ds(16, 16)] * gk
                    rows[j, pl.ds(32, 16)] = rows[j, pl.ds(32, 16)] * gk
                    rows[j, pl.ds(48, 16)] = rows[j, pl.ds(48, 16)] * gk
                    rows[j, pl.ds(64, 16)] = rows[j, pl.ds(64, 16)] * exk

            pltpu.sync_copy(rows, acc.at[dst_v.at[b]], add=True)

    plsc.subcore_barrier()
    pltpu.sync_copy(acc.at[pl.ds(s * RPT, RPT), :],
                    out_hbm.at[c, pl.ds(s * RPT, RPT), :])


def _sc_edge(htab, sd, src, dst, ew):
    mesh = plsc.VectorSubcoreMesh(core_axis_name="c", subcore_axis_name="s")
    k = pl.kernel(
        _sc_edge_body,
        out_type=jax.ShapeDtypeStruct((NC, N, WD), jnp.float32),
        mesh=mesh,
        compiler_params=_compiler_params_sc(),
        scratch_types=[
            pltpu.VMEM_SHARED((N, WD), jnp.float32),
            pltpu.VMEM((2 * N,), jnp.float32),
            pltpu.VMEM((CB, EB), jnp.int32),
            pltpu.VMEM((CB, EB), jnp.int32),
            pltpu.VMEM((CB, EB), jnp.float32),
            pltpu.VMEM((EB, WD), jnp.float32),
            pltpu.SemaphoreType.DMA,
            pltpu.SemaphoreType.DMA,
        ],
    )
    return k(htab, sd, src, dst, ew)


def _tc1_body(x_ref, w_ref, as_ref, ad_ref, htab_ref, sd_ref):
    h = jnp.dot(x_ref[...], w_ref[...], preferred_element_type=jnp.float32)
    one = jnp.ones((_ROW_BLK, 1), jnp.float32)
    zer = jnp.zeros((_ROW_BLK, WD - H - 1), jnp.float32)
    htab_ref[...] = jnp.concatenate([h, one, zer], axis=1)
    sv = h @ as_ref[...]
    dv = h @ ad_ref[...]
    sd_ref[...] = jnp.stack([sv, dv], axis=1)


def _tc1(x, W1, a_src1, a_dst1):
    return pl.pallas_call(
        _tc1_body,
        grid=(_NBLK,),
        in_specs=[
            pl.BlockSpec((_ROW_BLK, D), lambda i: (i, 0)),
            pl.BlockSpec((D, H), lambda i: (0, 0)),
            pl.BlockSpec((H,), lambda i: (0,)),
            pl.BlockSpec((H,), lambda i: (0,)),
        ],
        out_specs=[
            pl.BlockSpec((_ROW_BLK, WD), lambda i: (i, 0)),
            pl.BlockSpec((_ROW_BLK, 2), lambda i: (i, 0)),
        ],
        out_shape=[
            jax.ShapeDtypeStruct((N, WD), jnp.float32),
            jax.ShapeDtypeStruct((N, 2), jnp.float32),
        ],
    )(x, W1, a_src1, a_dst1)


def _fuse_layer(acc_blk, htab_blk, sd_blk, bias):
    sv = sd_blk[:, 0]
    dv = sd_blk[:, 1]
    a = sv + dv
    a = jnp.where(a >= 0, a, a * jnp.float32(0.2))
    ex = jnp.exp(a)
    num = (acc_blk[0, :, :H] + acc_blk[1, :, :H]
           + ex[:, None] * htab_blk[:, :H])
    den = acc_blk[0, :, H] + acc_blk[1, :, H] + ex
    return num / (den + 1e-16)[:, None] + bias


def _tc2_body(acc_ref, htab_ref, sd_ref, b1_ref, w_ref, as_ref, ad_ref,
              htab2_ref, sd2_ref):
    h1 = jax.nn.relu(_fuse_layer(acc_ref[...], htab_ref[...], sd_ref[...],
                                 b1_ref[...]))
    h2 = jnp.dot(h1, w_ref[...], preferred_element_type=jnp.float32)
    one = jnp.ones((_ROW_BLK, 1), jnp.float32)
    zer = jnp.zeros((_ROW_BLK, WD - H - 1), jnp.float32)
    htab2_ref[...] = jnp.concatenate([h2, one, zer], axis=1)
    sv = h2 @ as_ref[...]
    dv = h2 @ ad_ref[...]
    sd2_ref[...] = jnp.stack([sv, dv], axis=1)


def _tc2(acc1, htab1, sd1, b1, W2, a_src2, a_dst2):
    return pl.pallas_call(
        _tc2_body,
        grid=(_NBLK,),
        in_specs=[
            pl.BlockSpec((NC, _ROW_BLK, WD), lambda i: (0, i, 0)),
            pl.BlockSpec((_ROW_BLK, WD), lambda i: (i, 0)),
            pl.BlockSpec((_ROW_BLK, 2), lambda i: (i, 0)),
            pl.BlockSpec((H,), lambda i: (0,)),
            pl.BlockSpec((H, H), lambda i: (0, 0)),
            pl.BlockSpec((H,), lambda i: (0,)),
            pl.BlockSpec((H,), lambda i: (0,)),
        ],
        out_specs=[
            pl.BlockSpec((_ROW_BLK, WD), lambda i: (i, 0)),
            pl.BlockSpec((_ROW_BLK, 2), lambda i: (i, 0)),
        ],
        out_shape=[
            jax.ShapeDtypeStruct((N, WD), jnp.float32),
            jax.ShapeDtypeStruct((N, 2), jnp.float32),
        ],
    )(acc1, htab1, sd1, b1, W2, a_src2, a_dst2)


def _tc3_body(acc_ref, htab_ref, sd_ref, b2_ref, batch_ref,
              wl_ref, bl_ref, wc_ref, bc_ref, o_ref, pool_ref):
    i = pl.program_id(0)

    @pl.when(i == 0)
    def _():
        pool_ref[...] = jnp.zeros((G, 2 * H), jnp.float32)

    h2 = _fuse_layer(acc_ref[...], htab_ref[...], sd_ref[...], b2_ref[...])
    batch = batch_ref[0, 0, :]
    gids = lax.broadcasted_iota(jnp.int32, (G, _ROW_BLK), 0)
    mask = (gids == batch[None, :]).astype(jnp.float32)
    he = jnp.concatenate([h2, jnp.ones((_ROW_BLK, H), jnp.float32)], axis=1)
    pool_ref[...] += jnp.dot(mask, he, preferred_element_type=jnp.float32)

    @pl.when(i == _NBLK - 1)
    def _():
        pooled = pool_ref[:, :H] / jnp.maximum(pool_ref[:, H:H + 1], 1.0)
        z = jax.nn.relu(pooled @ wl_ref[...] + bl_ref[...])
        logits = z @ wc_ref[...] + bc_ref[...]
        o_ref[...] = jax.nn.log_softmax(logits, axis=1)


def _tc3(acc2, htab2, sd2, b2, batch3, Wlin, blin, Wcls, bcls):
    return pl.pallas_call(
        _tc3_body,
        grid=(_NBLK,),
        in_specs=[
            pl.BlockSpec((NC, _ROW_BLK, WD), lambda i: (0, i, 0)),
            pl.BlockSpec((_ROW_BLK, WD), lambda i: (i, 0)),
            pl.BlockSpec((_ROW_BLK, 2), lambda i: (i, 0)),
            pl.BlockSpec((H,), lambda i: (0,)),
            pl.BlockSpec((1, 1, _ROW_BLK), lambda i: (i, 0, 0)),
            pl.BlockSpec((H, H // 2), lambda i: (0, 0)),
            pl.BlockSpec((H // 2,), lambda i: (0,)),
            pl.BlockSpec((H // 2, C), lambda i: (0, 0)),
            pl.BlockSpec((C,), lambda i: (0,)),
        ],
        out_specs=pl.BlockSpec((G, C), lambda i: (0, 0)),
        out_shape=jax.ShapeDtypeStruct((G, C), jnp.float32),
        scratch_shapes=[pltpu.VMEM((G, 2 * H), jnp.float32)],
    )(acc2, htab2, sd2, b2, batch3, Wlin, blin, Wcls, bcls)


def kernel(x, edge_index, edge_weight, batch, W1, a_src1, a_dst1, b1,
           W2, a_src2, a_dst2, b2, Wlin, blin, Wcls, bcls):
    src = edge_index[0].reshape(NW, NB, EB)
    dst = edge_index[1].reshape(NW, NB, EB)
    ew = edge_weight.reshape(NW, NB, EB)
    batch3 = batch.reshape(_NBLK, 1, _ROW_BLK)

    htab1, sd1 = _tc1(x, W1, a_src1, a_dst1)
    acc1 = _sc_edge(htab1, sd1.reshape(2 * N), src, dst, ew)
    htab2, sd2 = _tc2(acc1, htab1, sd1, b1, W2, a_src2, a_dst2)
    acc2 = _sc_edge(htab2, sd2.reshape(2 * N), src, dst, ew)
    return _tc3(acc2, htab2, sd2, b2, batch3, Wlin, blin, Wcls, bcls)
